# baseline (speedup 1.0000x reference)
import jax
import jax.numpy as jnp
from jax.experimental import pallas as pl

N = 10000
B = 128


def _copy_body(x_ref, o_ref):
    o_ref[...] = x_ref[...]


def _gcn_conv(x, edge_index, W, b):
    src = edge_index[0]
    dst = edge_index[1]
    loop = jnp.arange(N, dtype=src.dtype)
    src = jnp.concatenate([src, loop])
    dst = jnp.concatenate([dst, loop])
    h = x @ W
    deg = jnp.zeros((N,), dtype=x.dtype).at[dst].add(1.0)
    dinv = jnp.where(deg > 0, jax.lax.rsqrt(jnp.maximum(deg, 1e-12)), 0.0)
    norm = dinv[src] * dinv[dst]
    msg = h[src] * norm[:, None]
    out = jnp.zeros((N, W.shape[1]), dtype=x.dtype).at[dst].add(msg)
    return out + b


def kernel(x, edge_index, batch, W1, b1, W2, b2, Wfc1, bfc1, Wfc2, bfc2):
    h = jax.nn.relu(_gcn_conv(x, edge_index, W1, b1))
    h = jax.nn.relu(_gcn_conv(h, edge_index, W2, b2))
    sums = jax.ops.segment_sum(h, batch, num_segments=B)
    counts = jax.ops.segment_sum(jnp.ones((N, 1), dtype=h.dtype), batch, num_segments=B)
    pooled = sums / jnp.maximum(counts, 1.0)
    h = jax.nn.relu(pooled @ Wfc1 + bfc1)
    out = h @ Wfc2 + bfc2
    return pl.pallas_call(
        _copy_body, out_shape=jax.ShapeDtypeStruct(out.shape, out.dtype)
    )(out)


# speedup vs baseline: 30.4598x; 30.4598x over previous
import functools

import jax
import jax.numpy as jnp
from jax import lax
from jax.experimental import pallas as pl
from jax.experimental.pallas import tpu as pltpu
from jax.experimental.pallas import tpu_sc as plsc

N = 10000
E = 320000
D = 128
B = 128
C = 10

NC = 2
NS = 16
NW = NC * NS
CB = 128
CPT = (E + NW * CB - 1) // (NW * CB)
E_PAD = NW * CPT * CB
N_PAD = 10240
RPT = N_PAD // NS


def _mesh():
    return plsc.VectorSubcoreMesh(core_axis_name="c", subcore_axis_name="s")


def _deg_body(dst_hbm, zeros_hbm, ones_hbm, out_hbm, idx_v, ones_v, deg_sh):
    c = lax.axis_index("c")
    s = lax.axis_index("s")
    w = c * NS + s
    pltpu.sync_copy(zeros_hbm.at[pl.ds(s * RPT, RPT)],
                    deg_sh.at[pl.ds(s * RPT, RPT)])
    pltpu.sync_copy(ones_hbm, ones_v)
    pltpu.sync_copy(dst_hbm.at[w], idx_v)
    plsc.subcore_barrier()

    def body(j, carry):
        pltpu.sync_copy(ones_v, deg_sh.at[idx_v.at[j]], add=True)
        return carry

    lax.fori_loop(0, CPT, body, 0)
    plsc.subcore_barrier()
    pltpu.sync_copy(deg_sh.at[pl.ds(s * RPT, RPT)],
                    out_hbm.at[c, pl.ds(s * RPT, RPT)])


_deg_sc = pl.kernel(
    _deg_body,
    out_type=jax.ShapeDtypeStruct((NC, N_PAD, 16), jnp.float32),
    mesh=_mesh(),
    scratch_types=[
        pltpu.VMEM((CPT, CB), jnp.int32),
        pltpu.VMEM((CB, 16), jnp.float32),
        pltpu.VMEM_SHARED((N_PAD, 16), jnp.float32),
    ],
    compiler_params=pltpu.CompilerParams(use_tc_tiling_on_sc=False),
)


def _agg_body(F, h_hbm, src_hbm, dst_hbm, zeros_hbm, out_hbm,
              isrc_v, idst_v, rows_v, agg_sh, sem):
    del F
    c = lax.axis_index("c")
    s = lax.axis_index("s")
    w = c * NS + s
    pltpu.sync_copy(zeros_hbm.at[pl.ds(s * RPT, RPT)],
                    agg_sh.at[pl.ds(s * RPT, RPT)])
    pltpu.sync_copy(src_hbm.at[w], isrc_v)
    pltpu.sync_copy(dst_hbm.at[w], idst_v)
    plsc.subcore_barrier()

    def body(j, carry):
        pltpu.async_copy(h_hbm.at[isrc_v.at[j]], rows_v, sem).wait()
        pltpu.sync_copy(rows_v, agg_sh.at[idst_v.at[j]], add=True)
        return carry

    lax.fori_loop(0, CPT, body, 0)
    plsc.subcore_barrier()
    pltpu.sync_copy(agg_sh.at[pl.ds(s * RPT, RPT)],
                    out_hbm.at[c, pl.ds(s * RPT, RPT)])


def _make_agg(F):
    return pl.kernel(
        functools.partial(_agg_body, F),
        out_type=jax.ShapeDtypeStruct((NC, N_PAD, F), jnp.float32),
        mesh=_mesh(),
        scratch_types=[
            pltpu.VMEM((CPT, CB), jnp.int32),
            pltpu.VMEM((CPT, CB), jnp.int32),
            pltpu.VMEM((CB, F), jnp.float32),
            pltpu.VMEM_SHARED((N_PAD, F), jnp.float32),
            pltpu.SemaphoreType.DMA,
        ],
        compiler_params=pltpu.CompilerParams(use_tc_tiling_on_sc=False),
    )


_agg16 = _make_agg(16)
_agg32 = _make_agg(32)


def _tc1_body(x_ref, w1_ref, degp_ref, h1s_ref, dinv_ref):
    deg = degp_ref[0][:, :1] + degp_ref[1][:, :1] + 1.0
    dinv = lax.rsqrt(deg)
    dinv_ref[...] = dinv
    h = jnp.dot(x_ref[...], w1_ref[...], preferred_element_type=jnp.float32)
    h1s_ref[...] = h * dinv[:N]


def _tc2_body(agg_ref, h1s_ref, dinv_ref, b1_ref, w2_ref, h2s_ref):
    dinv = dinv_ref[...][:N]
    z = (agg_ref[0][:N] + agg_ref[1][:N] + h1s_ref[...]) * dinv + b1_ref[...]
    z = jnp.maximum(z, 0.0)
    h2s_ref[...] = jnp.dot(z, w2_ref[...],
                           preferred_element_type=jnp.float32) * dinv


def _tc3_body(agg_ref, h2s_ref, dinv_ref, b2_ref, batch_ref,
              wfc1_ref, bfc1_ref, wfc2_ref, bfc2_ref, out_ref):
    dinv = dinv_ref[...][:N]
    z = (agg_ref[0][:N] + agg_ref[1][:N] + h2s_ref[...]) * dinv + b2_ref[...]
    z = jnp.maximum(z, 0.0)
    oh = (batch_ref[...] == lax.broadcasted_iota(jnp.int32, (B, N), 0))
    oh = oh.astype(jnp.float32)
    sums = jnp.dot(oh, z, preferred_element_type=jnp.float32)
    counts = jnp.sum(oh, axis=1, keepdims=True)
    pooled = sums / jnp.maximum(counts, 1.0)
    hfc = jnp.maximum(
        jnp.dot(pooled, wfc1_ref[...], preferred_element_type=jnp.float32)
        + bfc1_ref[...], 0.0)
    out_ref[...] = jnp.dot(hfc, wfc2_ref[...],
                           preferred_element_type=jnp.float32) + bfc2_ref[...]


_tc1 = pl.pallas_call(
    _tc1_body,
    out_shape=(jax.ShapeDtypeStruct((N, 16), jnp.float32),
               jax.ShapeDtypeStruct((N_PAD, 1), jnp.float32)))

_tc2 = pl.pallas_call(
    _tc2_body, out_shape=jax.ShapeDtypeStruct((N, 32), jnp.float32))

_tc3 = pl.pallas_call(
    _tc3_body, out_shape=jax.ShapeDtypeStruct((B, C), jnp.float32))


def kernel(x, edge_index, batch, W1, b1, W2, b2, Wfc1, bfc1, Wfc2, bfc2):
    src = edge_index[0]
    dst = edge_index[1]
    pad = E_PAD - E
    src_p = jnp.concatenate(
        [src, jnp.zeros((pad,), jnp.int32)]).reshape(NW, CPT, CB)
    dst_p = jnp.concatenate(
        [dst, jnp.full((pad,), N, jnp.int32)]).reshape(NW, CPT, CB)

    zeros1 = jnp.zeros((N_PAD, 16), jnp.float32)
    ones1 = jnp.ones((CB, 16), jnp.float32)
    degp = _deg_sc(dst_p, zeros1, ones1)

    h1s, dinv = _tc1(x, W1, degp)

    zeros16 = jnp.zeros((N_PAD, 16), jnp.float32)
    agg1 = _agg16(h1s, src_p, dst_p, zeros16)

    h2s = _tc2(agg1, h1s, dinv, b1.reshape(1, 16), W2)

    zeros32 = jnp.zeros((N_PAD, 32), jnp.float32)
    agg2 = _agg32(h2s, src_p, dst_p, zeros32)

    return _tc3(agg2, h2s, dinv, b2.reshape(1, 32), batch.reshape(1, N),
                Wfc1, bfc1.reshape(1, 64), Wfc2, bfc2.reshape(1, C))


# speedup vs baseline: 30.4787x; 1.0006x over previous
import functools

import jax
import jax.numpy as jnp
from jax import lax
from jax.experimental import pallas as pl
from jax.experimental.pallas import tpu as pltpu
from jax.experimental.pallas import tpu_sc as plsc

N = 10000
E = 320000
D = 128
B = 128
C = 10

NC = 2
NS = 16
NW = NC * NS
CB = 128
CPT = 80
E_PAD = NW * CPT * CB
NB = 2
N_PAD = 10240
RPT = N_PAD // NS


def _mesh():
    return plsc.VectorSubcoreMesh(core_axis_name="c", subcore_axis_name="s")


def _deg_body(dst_hbm, zeros_hbm, ones_hbm, out_hbm, idx_v, ones_v, deg_sh):
    c = lax.axis_index("c")
    s = lax.axis_index("s")
    w = c * NS + s
    pltpu.sync_copy(zeros_hbm.at[pl.ds(s * RPT, RPT)],
                    deg_sh.at[pl.ds(s * RPT, RPT)])
    pltpu.sync_copy(ones_hbm, ones_v)
    pltpu.sync_copy(dst_hbm.at[w], idx_v)
    plsc.subcore_barrier()

    def body(j, carry):
        pltpu.sync_copy(ones_v, deg_sh.at[idx_v.at[j]], add=True)
        return carry

    lax.fori_loop(0, CPT, body, 0)
    plsc.subcore_barrier()
    pltpu.sync_copy(deg_sh.at[pl.ds(s * RPT, RPT)],
                    out_hbm.at[c, pl.ds(s * RPT, RPT)])


_deg_sc = pl.kernel(
    _deg_body,
    out_type=jax.ShapeDtypeStruct((NC, N_PAD, 16), jnp.float32),
    mesh=_mesh(),
    scratch_types=[
        pltpu.VMEM((CPT, CB), jnp.int32),
        pltpu.VMEM((CB, 16), jnp.float32),
        pltpu.VMEM_SHARED((N_PAD, 16), jnp.float32),
    ],
    compiler_params=pltpu.CompilerParams(use_tc_tiling_on_sc=False),
)


def _agg_body(F, h_hbm, src_hbm, dst_hbm, zeros_hbm, out_hbm,
              isrc_v, idst_v, rows_v, agg_sh, sem0, sem1):
    c = lax.axis_index("c")
    s = lax.axis_index("s")
    w = c * NS + s
    sems = (sem0, sem1)
    pltpu.sync_copy(zeros_hbm.at[pl.ds(s * RPT, RPT)],
                    agg_sh.at[pl.ds(s * RPT, RPT)])
    pltpu.sync_copy(src_hbm.at[w], isrc_v)
    pltpu.sync_copy(dst_hbm.at[w], idst_v)
    plsc.subcore_barrier()

    def start_gather(j, b):
        pltpu.async_copy(h_hbm.at[isrc_v.at[j]], rows_v.at[b], sems[b])

    def wait_gather(j, b):
        pltpu.make_async_copy(h_hbm.at[isrc_v.at[j]], rows_v.at[b],
                              sems[b]).wait()

    start_gather(0, 0)

    def body(i, carry):
        for b in range(NB):
            j = i * NB + b
            jn = j + 1

            @pl.when(jn < CPT)
            def _():
                start_gather(jn, (b + 1) % NB)

            wait_gather(j, b)
            pltpu.sync_copy(rows_v.at[b], agg_sh.at[idst_v.at[j]], add=True)
        return carry

    lax.fori_loop(0, CPT // NB, body, 0)
    plsc.subcore_barrier()
    pltpu.sync_copy(agg_sh.at[pl.ds(s * RPT, RPT)],
                    out_hbm.at[c, pl.ds(s * RPT, RPT)])


def _make_agg(F):
    return pl.kernel(
        functools.partial(_agg_body, F),
        out_type=jax.ShapeDtypeStruct((NC, N_PAD, F), jnp.float32),
        mesh=_mesh(),
        scratch_types=[
            pltpu.VMEM((CPT, CB), jnp.int32),
            pltpu.VMEM((CPT, CB), jnp.int32),
            pltpu.VMEM((NB, CB, F), jnp.float32),
            pltpu.VMEM_SHARED((N_PAD, F), jnp.float32),
            pltpu.SemaphoreType.DMA,
            pltpu.SemaphoreType.DMA,
        ],
        compiler_params=pltpu.CompilerParams(use_tc_tiling_on_sc=False),
    )


_agg16 = _make_agg(16)
_agg32 = _make_agg(32)


def _tc1_body(x_ref, w1_ref, degp_ref, h1s_ref, dinv_ref):
    deg = degp_ref[0][:, :1] + degp_ref[1][:, :1] + 1.0
    dinv = lax.rsqrt(deg)
    dinv_ref[...] = dinv
    h = jnp.dot(x_ref[...], w1_ref[...], preferred_element_type=jnp.float32)
    h1s_ref[...] = h * dinv[:N]


def _tc2_body(agg_ref, h1s_ref, dinv_ref, b1_ref, w2_ref, h2s_ref):
    dinv = dinv_ref[...][:N]
    z = (agg_ref[0][:N] + agg_ref[1][:N] + h1s_ref[...]) * dinv + b1_ref[...]
    z = jnp.maximum(z, 0.0)
    h2s_ref[...] = jnp.dot(z, w2_ref[...],
                           preferred_element_type=jnp.float32) * dinv


def _tc3_body(agg_ref, h2s_ref, dinv_ref, b2_ref, batch_ref,
              wfc1_ref, bfc1_ref, wfc2_ref, bfc2_ref, out_ref):
    dinv = dinv_ref[...][:N]
    z = (agg_ref[0][:N] + agg_ref[1][:N] + h2s_ref[...]) * dinv + b2_ref[...]
    z = jnp.maximum(z, 0.0)
    oh = (batch_ref[...] == lax.broadcasted_iota(jnp.int32, (B, N), 0))
    oh = oh.astype(jnp.float32)
    sums = jnp.dot(oh, z, preferred_element_type=jnp.float32)
    counts = jnp.sum(oh, axis=1, keepdims=True)
    pooled = sums / jnp.maximum(counts, 1.0)
    hfc = jnp.maximum(
        jnp.dot(pooled, wfc1_ref[...], preferred_element_type=jnp.float32)
        + bfc1_ref[...], 0.0)
    out_ref[...] = jnp.dot(hfc, wfc2_ref[...],
                           preferred_element_type=jnp.float32) + bfc2_ref[...]


_tc1 = pl.pallas_call(
    _tc1_body,
    out_shape=(jax.ShapeDtypeStruct((N, 16), jnp.float32),
               jax.ShapeDtypeStruct((N_PAD, 1), jnp.float32)))

_tc2 = pl.pallas_call(
    _tc2_body, out_shape=jax.ShapeDtypeStruct((N, 32), jnp.float32))

_tc3 = pl.pallas_call(
    _tc3_body, out_shape=jax.ShapeDtypeStruct((B, C), jnp.float32))


def kernel(x, edge_index, batch, W1, b1, W2, b2, Wfc1, bfc1, Wfc2, bfc2):
    src = edge_index[0]
    dst = edge_index[1]
    pad = E_PAD - E
    src_p = jnp.concatenate(
        [src, jnp.zeros((pad,), jnp.int32)]).reshape(NW, CPT, CB)
    dst_p = jnp.concatenate(
        [dst, jnp.full((pad,), N, jnp.int32)]).reshape(NW, CPT, CB)

    zeros1 = jnp.zeros((N_PAD, 16), jnp.float32)
    ones1 = jnp.ones((CB, 16), jnp.float32)
    degp = _deg_sc(dst_p, zeros1, ones1)

    h1s, dinv = _tc1(x, W1, degp)

    zeros16 = jnp.zeros((N_PAD, 16), jnp.float32)
    agg1 = _agg16(h1s, src_p, dst_p, zeros16)

    h2s = _tc2(agg1, h1s, dinv, b1.reshape(1, 16), W2)

    zeros32 = jnp.zeros((N_PAD, 32), jnp.float32)
    agg2 = _agg32(h2s, src_p, dst_p, zeros32)

    return _tc3(agg2, h2s, dinv, b2.reshape(1, 32), batch.reshape(1, N),
                Wfc1, bfc1.reshape(1, 64), Wfc2, bfc2.reshape(1, C))
